# baseline (speedup 1.0000x reference)
import functools

import jax
import jax.numpy as jnp
from jax import lax
from jax.experimental import pallas as pl
from jax.experimental.pallas import tpu as pltpu
from jax.experimental.pallas import tpu_sc as plsc

EMB_DIM = 128
BATCH = 16384

_info = plsc.get_sparse_core_info()
_NC, _NS = _info.num_cores, _info.num_subcores
_NW = _NC * _NS
_BPW = BATCH // _NW

_mesh = plsc.VectorSubcoreMesh(core_axis_name="c", subcore_axis_name="s")


@functools.partial(
    pl.kernel,
    mesh=_mesh,
    out_type=jax.ShapeDtypeStruct((BATCH, EMB_DIM), jnp.float32),
    scratch_types=[
        pltpu.VMEM((_BPW,), jnp.int32),
        pltpu.VMEM((_BPW, EMB_DIM), jnp.float32),
        pltpu.SemaphoreType.DMA,
    ],
)
def _lookup(idx_hbm, table_hbm, out_hbm, idx_v, rows_v, sem):
    wid = lax.axis_index("s") * _NC + lax.axis_index("c")
    base = wid * _BPW
    pltpu.sync_copy(idx_hbm.at[pl.ds(base, _BPW)], idx_v)
    pltpu.async_copy(table_hbm.at[idx_v], rows_v, sem).wait()
    pltpu.sync_copy(rows_v, out_hbm.at[pl.ds(base, _BPW)])


def kernel(x, table):
    idx = jnp.reshape(x, (BATCH,)).astype(jnp.int32)
    return _lookup(idx, table)


# speedup vs baseline: 5.6692x; 5.6692x over previous
import functools

import jax
import jax.numpy as jnp
from jax import lax
from jax.experimental import pallas as pl
from jax.experimental.pallas import tpu as pltpu
from jax.experimental.pallas import tpu_sc as plsc

EMB_DIM = 128
BATCH = 16384
_LANES = 16
_CHUNKS = EMB_DIM // _LANES

_info = plsc.get_sparse_core_info()
_NC, _NS = _info.num_cores, _info.num_subcores
_NW = _NC * _NS
_BPW = BATCH // _NW
_NBUF = 4
_ROWS_PER_BUF = _BPW // _NBUF

_mesh = plsc.VectorSubcoreMesh(core_axis_name="c", subcore_axis_name="s")


@functools.partial(
    pl.kernel,
    mesh=_mesh,
    out_type=jax.ShapeDtypeStruct((BATCH, EMB_DIM), jnp.float32),
    scratch_types=[
        pltpu.VMEM((_BPW,), jnp.int32),
        pltpu.VMEM((3, EMB_DIM), jnp.float32),
        pltpu.VMEM((_BPW, EMB_DIM), jnp.float32),
        pltpu.SemaphoreType.DMA,
    ],
)
def _lookup(idx_hbm, table_hbm, out_hbm, idx_v, table_v, rows_v, sem):
    wid = lax.axis_index("s") * _NC + lax.axis_index("c")
    base = wid * _BPW
    pltpu.sync_copy(table_hbm, table_v)
    pltpu.sync_copy(idx_hbm.at[pl.ds(base, _BPW)], idx_v)

    copies = []
    groups_per_buf = _ROWS_PER_BUF // _LANES
    for buf in range(_NBUF):
        def group_body(g, carry):
            b0 = g * _LANES
            idx16 = idx_v[pl.ds(b0, _LANES)]
            for r in range(_LANES):
                row = idx16[r]
                for j in range(_CHUNKS):
                    sl = pl.ds(j * _LANES, _LANES)
                    rows_v[b0 + r, sl] = table_v[row, sl]
            return carry

        lax.fori_loop(buf * groups_per_buf, (buf + 1) * groups_per_buf,
                      group_body, 0)
        copies.append(pltpu.async_copy(
            rows_v.at[pl.ds(buf * _ROWS_PER_BUF, _ROWS_PER_BUF)],
            out_hbm.at[pl.ds(base + buf * _ROWS_PER_BUF, _ROWS_PER_BUF)],
            sem))
    for cp in copies:
        cp.wait()


def kernel(x, table):
    idx = jnp.reshape(x, (BATCH,)).astype(jnp.int32)
    return _lookup(idx, table)


# speedup vs baseline: 7.7409x; 1.3654x over previous
import functools

import jax
import jax.numpy as jnp
from jax import lax
from jax.experimental import pallas as pl
from jax.experimental.pallas import tpu as pltpu
from jax.experimental.pallas import tpu_sc as plsc

EMB_DIM = 128
BATCH = 16384
_LANES = 16
_REP = 1024

_info = plsc.get_sparse_core_info()
_NC, _NS = _info.num_cores, _info.num_subcores
_NW = _NC * _NS
_BPW = BATCH // _NW
_NBUF = 4
_ROWS_PER_BUF = _BPW // _NBUF

_mesh = plsc.VectorSubcoreMesh(core_axis_name="c", subcore_axis_name="s")


@functools.partial(
    pl.kernel,
    mesh=_mesh,
    out_type=jax.ShapeDtypeStruct((BATCH, EMB_DIM), jnp.float32),
    scratch_types=[
        pltpu.VMEM((_BPW,), jnp.int32),
        pltpu.VMEM((_BPW, EMB_DIM), jnp.float32),
        pltpu.SemaphoreType.DMA,
        pltpu.SemaphoreType.DMA,
    ],
)
def _lookup(idx_hbm, table_hbm, out_hbm, idx_v, rows_v, gsem, osem):
    wid = lax.axis_index("s") * _NC + lax.axis_index("c")
    base = wid * _BPW
    pltpu.sync_copy(idx_hbm.at[pl.ds(base, _BPW)], idx_v)

    lane3 = lax.iota(jnp.int32, _LANES) * 3
    for g in range(_BPW // _LANES):
        b0 = g * _LANES
        slot0 = ((base + b0) % _REP) * 3
        idx_v[pl.ds(b0, _LANES)] = idx_v[pl.ds(b0, _LANES)] + (slot0 + lane3)

    out_copies = []
    for buf in range(_NBUF):
        rsl = pl.ds(buf * _ROWS_PER_BUF, _ROWS_PER_BUF)
        pltpu.async_copy(table_hbm.at[idx_v.at[rsl]], rows_v.at[rsl],
                         gsem).wait()
        out_copies.append(pltpu.async_copy(
            rows_v.at[rsl],
            out_hbm.at[pl.ds(base + buf * _ROWS_PER_BUF, _ROWS_PER_BUF)],
            osem))
    for cp in out_copies:
        cp.wait()


def kernel(x, table):
    idx = jnp.reshape(x, (BATCH,)).astype(jnp.int32)
    table_rep = jnp.tile(table, (_REP, 1))
    return _lookup(idx, table_rep)
